# baseline (speedup 1.0000x reference)
import functools

import jax
import jax.numpy as jnp
from jax import lax
from jax.experimental import pallas as pl
from jax.experimental.pallas import tpu as pltpu, tpu_sc as plsc

BATCH = 4096
HIST = 50
EMBED = 128

NUM_CORES = 2
NUM_SUBCORES = 16
NUM_WORKERS = NUM_CORES * NUM_SUBCORES

BPW = BATCH // NUM_WORKERS
NBUF = 5
NLAPS = HIST // NBUF

_mesh = plsc.VectorSubcoreMesh(core_axis_name="c", subcore_axis_name="s")


@functools.partial(
    pl.kernel,
    mesh=_mesh,
    out_type=jax.ShapeDtypeStruct((HIST, BATCH, EMBED), jnp.float32),
    scratch_types=(
        [pltpu.VMEM((HIST, BPW), jnp.int32)]
        + [pltpu.VMEM((BPW, EMBED), jnp.float32) for _ in range(NBUF)]
        + [pltpu.SemaphoreType.DMA for _ in range(2 * NBUF)]
    ),
)
def _gather_sc(idx_hbm, table_hbm, out_hbm, idx_v, *rest):
    rows = rest[:NBUF]
    gsem = rest[NBUF : 2 * NBUF]
    ssem = rest[2 * NBUF : 3 * NBUF]

    wid = lax.axis_index("s") * NUM_CORES + lax.axis_index("c")
    bbase = wid * BPW
    pltpu.sync_copy(idx_hbm.at[:, pl.ds(bbase, BPW)], idx_v)

    def gather_start(b, h):
        pltpu.async_copy(table_hbm.at[idx_v.at[h]], rows[b], gsem[b])

    def gather_wait(b):
        pltpu.make_async_copy(table_hbm.at[idx_v.at[0]], rows[b], gsem[b]).wait()

    def store_start(b, h):
        pltpu.async_copy(rows[b], out_hbm.at[h, pl.ds(bbase, BPW)], ssem[b])

    def store_wait(b):
        pltpu.make_async_copy(rows[b], out_hbm.at[0, pl.ds(bbase, BPW)], ssem[b]).wait()

    for b in range(NBUF):
        gather_start(b, b)

    def lap(g, carry):
        for b in range(NBUF):
            gather_wait(b)
            store_start(b, g * NBUF + b)
        for b in range(NBUF):
            store_wait(b)
            gather_start(b, g * NBUF + b + NBUF)
        return carry

    lax.fori_loop(0, NLAPS - 1, lap, 0)

    for b in range(NBUF):
        gather_wait(b)
        store_start(b, HIST - NBUF + b)
    for b in range(NBUF):
        store_wait(b)


def kernel(indices, table):
    idx_t = indices.astype(jnp.int32).T
    out = _gather_sc(idx_t, table)
    return out.transpose(1, 0, 2)


# speedup vs baseline: 1.0273x; 1.0273x over previous
import functools

import jax
import jax.numpy as jnp
from jax import lax
from jax.experimental import pallas as pl
from jax.experimental.pallas import tpu as pltpu, tpu_sc as plsc

BATCH = 4096
HIST = 50
EMBED = 128

NUM_CORES = 2
NUM_SUBCORES = 16
NUM_WORKERS = NUM_CORES * NUM_SUBCORES

BPW = BATCH // NUM_WORKERS
CSPLIT = 2
CW = BPW // CSPLIT
NCHUNKS = HIST * CSPLIT
NBUF = 10
NLAPS = NCHUNKS // NBUF

_mesh = plsc.VectorSubcoreMesh(core_axis_name="c", subcore_axis_name="s")


@functools.partial(
    pl.kernel,
    mesh=_mesh,
    out_type=jax.ShapeDtypeStruct((HIST, BATCH, EMBED), jnp.float32),
    scratch_types=(
        [pltpu.VMEM((HIST, BPW), jnp.int32)]
        + [pltpu.VMEM((CW, EMBED), jnp.float32) for _ in range(NBUF)]
        + [pltpu.SemaphoreType.DMA for _ in range(2 * NBUF)]
    ),
)
def _gather_sc(idx_hbm, table_hbm, out_hbm, idx_v, *rest):
    rows = rest[:NBUF]
    gsem = rest[NBUF : 2 * NBUF]
    ssem = rest[2 * NBUF : 3 * NBUF]

    wid = lax.axis_index("s") * NUM_CORES + lax.axis_index("c")
    bbase = wid * BPW
    pltpu.sync_copy(idx_hbm.at[:, pl.ds(bbase, BPW)], idx_v)

    def gather_start(b, c):
        h = c // CSPLIT
        off = (c % CSPLIT) * CW
        pltpu.async_copy(table_hbm.at[idx_v.at[h, pl.ds(off, CW)]], rows[b], gsem[b])

    def gather_wait(b):
        pltpu.make_async_copy(
            table_hbm.at[idx_v.at[0, pl.ds(0, CW)]], rows[b], gsem[b]
        ).wait()

    def store_start(b, c):
        h = c // CSPLIT
        off = (c % CSPLIT) * CW
        pltpu.async_copy(rows[b], out_hbm.at[h, pl.ds(bbase + off, CW)], ssem[b])

    def store_wait(b):
        pltpu.make_async_copy(rows[b], out_hbm.at[0, pl.ds(bbase, CW)], ssem[b]).wait()

    for b in range(NBUF):
        gather_start(b, b)

    def lap(g, carry):
        for b in range(NBUF):
            c = g * NBUF + b
            gather_wait(b)
            store_start(b, c)
            store_wait(b)
            gather_start(b, c + NBUF)
        return carry

    lax.fori_loop(0, NLAPS - 1, lap, 0)

    for b in range(NBUF):
        gather_wait(b)
        store_start(b, NCHUNKS - NBUF + b)
    for b in range(NBUF):
        store_wait(b)


def kernel(indices, table):
    idx_t = indices.astype(jnp.int32).T
    out = _gather_sc(idx_t, table)
    return out.transpose(1, 0, 2)


# speedup vs baseline: 1.0383x; 1.0106x over previous
import functools

import jax
import jax.numpy as jnp
from jax import lax
from jax.experimental import pallas as pl
from jax.experimental.pallas import tpu as pltpu, tpu_sc as plsc

BATCH = 4096
HIST = 50
EMBED = 128

NUM_CORES = 2
NUM_SUBCORES = 16
NUM_WORKERS = NUM_CORES * NUM_SUBCORES

BPW = BATCH // NUM_WORKERS
CSPLIT = 1
CW = BPW // CSPLIT
NCHUNKS = HIST * CSPLIT
NBUF = 5
NLAPS = NCHUNKS // NBUF

_mesh = plsc.VectorSubcoreMesh(core_axis_name="c", subcore_axis_name="s")


@functools.partial(
    pl.kernel,
    mesh=_mesh,
    out_type=jax.ShapeDtypeStruct((HIST, BATCH, EMBED), jnp.float32),
    scratch_types=(
        [pltpu.VMEM((HIST, BPW), jnp.int32)]
        + [pltpu.VMEM((CW, EMBED), jnp.float32) for _ in range(NBUF)]
        + [pltpu.SemaphoreType.DMA for _ in range(2 * NBUF)]
    ),
)
def _gather_sc(idx_hbm, table_hbm, out_hbm, idx_v, *rest):
    rows = rest[:NBUF]
    gsem = rest[NBUF : 2 * NBUF]
    ssem = rest[2 * NBUF : 3 * NBUF]

    wid = lax.axis_index("s") * NUM_CORES + lax.axis_index("c")
    bbase = wid * BPW
    pltpu.sync_copy(idx_hbm.at[:, pl.ds(bbase, BPW)], idx_v)

    def gather_start(b, c):
        h = c // CSPLIT
        off = (c % CSPLIT) * CW
        pltpu.async_copy(table_hbm.at[idx_v.at[h, pl.ds(off, CW)]], rows[b], gsem[b])

    def gather_wait(b):
        pltpu.make_async_copy(
            table_hbm.at[idx_v.at[0, pl.ds(0, CW)]], rows[b], gsem[b]
        ).wait()

    def store_start(b, c):
        h = c // CSPLIT
        off = (c % CSPLIT) * CW
        pltpu.async_copy(rows[b], out_hbm.at[h, pl.ds(bbase + off, CW)], ssem[b])

    def store_wait(b):
        pltpu.make_async_copy(rows[b], out_hbm.at[0, pl.ds(bbase, CW)], ssem[b]).wait()

    for b in range(NBUF):
        gather_start(b, b)

    def lap(g, carry):
        for b in range(NBUF):
            c = g * NBUF + b
            gather_wait(b)
            store_start(b, c)
            store_wait(b)
            gather_start(b, c + NBUF)
        return carry

    lax.fori_loop(0, NLAPS - 1, lap, 0)

    for b in range(NBUF):
        gather_wait(b)
        store_start(b, NCHUNKS - NBUF + b)
    for b in range(NBUF):
        store_wait(b)


def kernel(indices, table):
    idx_t = indices.astype(jnp.int32).T
    out = _gather_sc(idx_t, table)
    return out.transpose(1, 0, 2)
